# baseline (speedup 1.0000x reference)
import functools

import jax
import jax.numpy as jnp
from jax.experimental import pallas as pl
from jax.experimental.pallas import tpu as pltpu

_E = 64
_K = 8
_COEF = 0.01
_NS = 4
_TC = 64


def _route_sub(x, w, b, rw_ref, se_ref):
    logits = jax.lax.dot_general(x, w, (((1,), (1,)), ((), ())),
                                 preferred_element_type=jnp.float32)
    logits = logits + b
    m = jnp.max(logits, axis=-1, keepdims=True)
    ex = jnp.exp(logits - m)
    scores = ex / jnp.sum(ex, axis=-1, keepdims=True)
    p_part = jnp.sum(scores, axis=0, keepdims=True)

    c_tot = None
    for c in range(x.shape[0] // _TC):
        s = scores[c * _TC:(c + 1) * _TC, :]
        iota = jax.lax.broadcasted_iota(jnp.int32, s.shape, 1)
        vals, idxs = [], []
        for _ in range(_K):
            mk = jnp.max(s, axis=-1, keepdims=True)
            ik = jnp.argmax(s, axis=-1, keepdims=True).astype(jnp.int32)
            vals.append(mk)
            idxs.append(ik)
            s = jnp.where(iota == ik, -1.0, s)
        topv = jnp.concatenate(vals, axis=-1)
        topi = jnp.concatenate(idxs, axis=-1)

        mm = jnp.max(topv, axis=-1, keepdims=True)
        e2 = jnp.exp(topv - mm)
        rw_ref[pl.ds(c * _TC, _TC), :] = e2 / jnp.sum(e2, axis=-1,
                                                      keepdims=True)
        se_ref[pl.ds(c * _TC, _TC), :] = topi

        c_part = jnp.sum((s < 0.0).astype(jnp.float32), axis=0,
                         keepdims=True)
        c_tot = c_part if c_tot is None else c_tot + c_part
    return p_part, c_tot


def _router_body(*refs, n_tokens):
    x_refs = refs[:_NS]
    w_ref, b_ref = refs[_NS], refs[_NS + 1]
    rw_refs = refs[_NS + 2:2 * _NS + 2]
    se_refs = refs[2 * _NS + 2:3 * _NS + 2]
    loss_ref = refs[3 * _NS + 2]
    acc_ref = refs[3 * _NS + 3]

    i = pl.program_id(0)
    n = pl.num_programs(0)
    w = w_ref[...]
    b = b_ref[...]

    p_tot, c_tot = None, None
    for s in range(_NS):
        p_part, c_part = _route_sub(x_refs[s][...], w, b,
                                    rw_refs[s], se_refs[s])
        p_tot = p_part if p_tot is None else p_tot + p_part
        c_tot = c_part if c_tot is None else c_tot + c_part

    @pl.when(i == 0)
    def _init():
        acc_ref[...] = jnp.zeros_like(acc_ref)

    acc_ref[0:1, :] += p_tot
    acc_ref[1:2, :] += c_tot

    @pl.when(i == n - 1)
    def _fin():
        p_i = acc_ref[0:1, :] / n_tokens
        f_i = acc_ref[1:2, :] / (n_tokens * _K)
        loss_ref[0, 0] = _COEF * _E * jnp.sum(f_i * p_i)


def kernel(hidden_states, W, b):
    B, S, H = hidden_states.shape
    N = B * S
    x = hidden_states.reshape(N, H)
    TH = min(256, N // _NS)
    nblk = N // (_NS * TH)

    def _in_spec(s):
        return pl.BlockSpec((TH, H), lambda i, s=s: (_NS * i + s, 0))

    outs = pl.pallas_call(
        functools.partial(_router_body, n_tokens=float(N)),
        grid=(nblk,),
        in_specs=[_in_spec(s) for s in range(_NS)] + [
            pl.BlockSpec((_E, H), lambda i: (0, 0)),
            pl.BlockSpec((1, _E), lambda i: (0, 0)),
        ],
        out_specs=[pl.BlockSpec((TH, _K), lambda i: (i, 0))
                   for _ in range(2 * _NS)] + [
            pl.BlockSpec((1, 1), lambda i: (0, 0), memory_space=pltpu.SMEM),
        ],
        out_shape=[jax.ShapeDtypeStruct((N // _NS, _K), jnp.float32)
                   for _ in range(_NS)] +
                  [jax.ShapeDtypeStruct((N // _NS, _K), jnp.int32)
                   for _ in range(_NS)] +
                  [jax.ShapeDtypeStruct((1, 1), jnp.float32)],
        scratch_shapes=[pltpu.VMEM((2, _E), jnp.float32)],
    )(*([x] * _NS), W, b.reshape(1, _E))

    rws = outs[:_NS]
    ses = outs[_NS:2 * _NS]
    loss = outs[2 * _NS]
    rw = jnp.stack([r.reshape(nblk, TH, _K) for r in rws],
                   axis=1).reshape(B, S, _K)
    se = jnp.stack([s.reshape(nblk, TH, _K) for s in ses],
                   axis=1).reshape(B, S, _K)
    return rw, se, loss[0, 0]


# speedup vs baseline: 1.1631x; 1.1631x over previous
import functools

import jax
import jax.numpy as jnp
from jax.experimental import pallas as pl
from jax.experimental.pallas import tpu as pltpu

_E = 64
_K = 8
_COEF = 0.01
_NS = 4


def _route_sub(x, w, b, rw_ref, se_ref):
    lt = jax.lax.dot_general(w, x, (((1,), (1,)), ((), ())),
                             preferred_element_type=jnp.float32)
    lt = lt + b
    m = jnp.max(lt, axis=0, keepdims=True)
    ex = jnp.exp(lt - m)
    scores = ex / jnp.sum(ex, axis=0, keepdims=True)
    p_part = jnp.sum(scores, axis=1, keepdims=True)

    iota = jax.lax.broadcasted_iota(jnp.int32, scores.shape, 0)
    s = scores
    vals, idxs = [], []
    for _ in range(_K):
        mk = jnp.max(s, axis=0, keepdims=True)
        ik = jnp.min(jnp.where(s == mk, iota, _E), axis=0,
                     keepdims=True)
        vals.append(mk)
        idxs.append(ik)
        s = jnp.where(iota == ik, -1.0, s)
    topv = jnp.concatenate(vals, axis=0)
    topi = jnp.concatenate(idxs, axis=0)

    mm = jnp.max(topv, axis=0, keepdims=True)
    e2 = jnp.exp(topv - mm)
    rw_ref[...] = e2 / jnp.sum(e2, axis=0, keepdims=True)
    se_ref[...] = topi

    c_part = jnp.sum((s < 0.0).astype(jnp.float32), axis=1,
                     keepdims=True)
    return p_part, c_part


def _router_body(*refs, n_tokens):
    x_refs = refs[:_NS]
    w_ref, b_ref = refs[_NS], refs[_NS + 1]
    rw_refs = refs[_NS + 2:2 * _NS + 2]
    se_refs = refs[2 * _NS + 2:3 * _NS + 2]
    loss_ref = refs[3 * _NS + 2]
    acc_ref = refs[3 * _NS + 3]

    i = pl.program_id(0)
    n = pl.num_programs(0)
    w = w_ref[...]
    b = b_ref[...]

    p_tot, c_tot = None, None
    for s in range(_NS):
        p_part, c_part = _route_sub(x_refs[s][...], w, b,
                                    rw_refs[s], se_refs[s])
        p_tot = p_part if p_tot is None else p_tot + p_part
        c_tot = c_part if c_tot is None else c_tot + c_part

    @pl.when(i == 0)
    def _init():
        acc_ref[...] = jnp.zeros_like(acc_ref)

    acc_ref[:, 0:1] += p_tot
    acc_ref[:, 1:2] += c_tot

    @pl.when(i == n - 1)
    def _fin():
        p_i = acc_ref[:, 0:1] / n_tokens
        f_i = acc_ref[:, 1:2] / (n_tokens * _K)
        loss_ref[0, 0] = _COEF * _E * jnp.sum(f_i * p_i)


def kernel(hidden_states, W, b):
    B, S, H = hidden_states.shape
    N = B * S
    x = hidden_states.reshape(N, H)
    TH = min(256, N // _NS)
    nblk = N // (_NS * TH)

    def _in_spec(s):
        return pl.BlockSpec((TH, H), lambda i, s=s: (_NS * i + s, 0))

    outs = pl.pallas_call(
        functools.partial(_router_body, n_tokens=float(N)),
        grid=(nblk,),
        in_specs=[_in_spec(s) for s in range(_NS)] + [
            pl.BlockSpec((_E, H), lambda i: (0, 0)),
            pl.BlockSpec((_E, 1), lambda i: (0, 0)),
        ],
        out_specs=[pl.BlockSpec((_K, TH), lambda i: (0, i))
                   for _ in range(2 * _NS)] + [
            pl.BlockSpec((1, 1), lambda i: (0, 0), memory_space=pltpu.SMEM),
        ],
        out_shape=[jax.ShapeDtypeStruct((_K, N // _NS), jnp.float32)
                   for _ in range(_NS)] +
                  [jax.ShapeDtypeStruct((_K, N // _NS), jnp.int32)
                   for _ in range(_NS)] +
                  [jax.ShapeDtypeStruct((1, 1), jnp.float32)],
        scratch_shapes=[pltpu.VMEM((_E, 2), jnp.float32)],
    )(*([x] * _NS), W, b.reshape(_E, 1))

    rws = outs[:_NS]
    ses = outs[_NS:2 * _NS]
    loss = outs[2 * _NS]
    rw = jnp.stack([r.reshape(_K, nblk, TH) for r in rws],
                   axis=2).transpose(1, 2, 3, 0).reshape(B, S, _K)
    se = jnp.stack([s.reshape(_K, nblk, TH) for s in ses],
                   axis=2).transpose(1, 2, 3, 0).reshape(B, S, _K)
    return rw, se, loss[0, 0]


# speedup vs baseline: 1.1705x; 1.0063x over previous
import functools

import jax
import jax.numpy as jnp
from jax.experimental import pallas as pl
from jax.experimental.pallas import tpu as pltpu

_E = 64
_K = 8
_COEF = 0.01
_NS = 2


def _route_sub(x, w, b, rw_ref, se_ref):
    lt = jax.lax.dot_general(w, x, (((1,), (1,)), ((), ())),
                             preferred_element_type=jnp.float32)
    lt = lt + b
    m = jnp.max(lt, axis=0, keepdims=True)
    ex = jnp.exp(lt - m)
    scores = ex / jnp.sum(ex, axis=0, keepdims=True)
    p_part = jnp.sum(scores, axis=1, keepdims=True)

    iota = jax.lax.broadcasted_iota(jnp.int32, scores.shape, 0)
    s = scores
    vals, idxs = [], []
    for _ in range(_K):
        mk = jnp.max(s, axis=0, keepdims=True)
        ik = jnp.min(jnp.where(s == mk, iota, _E), axis=0,
                     keepdims=True)
        vals.append(mk)
        idxs.append(ik)
        s = jnp.where(iota == ik, -1.0, s)
    topv = jnp.concatenate(vals, axis=0)
    topi = jnp.concatenate(idxs, axis=0)

    mm = jnp.max(topv, axis=0, keepdims=True)
    e2 = jnp.exp(topv - mm)
    rw_ref[...] = e2 / jnp.sum(e2, axis=0, keepdims=True)
    se_ref[...] = topi

    c_part = jnp.sum((s < 0.0).astype(jnp.float32), axis=1,
                     keepdims=True)
    return p_part, c_part


def _router_body(*refs, n_tokens):
    x_refs = refs[:_NS]
    w_ref, b_ref = refs[_NS], refs[_NS + 1]
    rw_refs = refs[_NS + 2:2 * _NS + 2]
    se_refs = refs[2 * _NS + 2:3 * _NS + 2]
    loss_ref = refs[3 * _NS + 2]
    acc_ref = refs[3 * _NS + 3]

    i = pl.program_id(0)
    n = pl.num_programs(0)
    w = w_ref[...]
    b = b_ref[...]

    p_tot, c_tot = None, None
    for s in range(_NS):
        p_part, c_part = _route_sub(x_refs[s][...], w, b,
                                    rw_refs[s], se_refs[s])
        p_tot = p_part if p_tot is None else p_tot + p_part
        c_tot = c_part if c_tot is None else c_tot + c_part

    @pl.when(i == 0)
    def _init():
        acc_ref[...] = jnp.zeros_like(acc_ref)

    acc_ref[:, 0:1] += p_tot
    acc_ref[:, 1:2] += c_tot

    @pl.when(i == n - 1)
    def _fin():
        p_i = acc_ref[:, 0:1] / n_tokens
        f_i = acc_ref[:, 1:2] / (n_tokens * _K)
        loss_ref[0, 0] = _COEF * _E * jnp.sum(f_i * p_i)


def kernel(hidden_states, W, b):
    B, S, H = hidden_states.shape
    N = B * S
    x = hidden_states.reshape(N, H)
    TH = min(512, N // _NS)
    nblk = N // (_NS * TH)

    def _in_spec(s):
        return pl.BlockSpec((TH, H), lambda i, s=s: (_NS * i + s, 0))

    outs = pl.pallas_call(
        functools.partial(_router_body, n_tokens=float(N)),
        grid=(nblk,),
        in_specs=[_in_spec(s) for s in range(_NS)] + [
            pl.BlockSpec((_E, H), lambda i: (0, 0)),
            pl.BlockSpec((_E, 1), lambda i: (0, 0)),
        ],
        out_specs=[pl.BlockSpec((_K, TH), lambda i: (0, i))
                   for _ in range(2 * _NS)] + [
            pl.BlockSpec((1, 1), lambda i: (0, 0), memory_space=pltpu.SMEM),
        ],
        out_shape=[jax.ShapeDtypeStruct((_K, N // _NS), jnp.float32)
                   for _ in range(_NS)] +
                  [jax.ShapeDtypeStruct((_K, N // _NS), jnp.int32)
                   for _ in range(_NS)] +
                  [jax.ShapeDtypeStruct((1, 1), jnp.float32)],
        scratch_shapes=[pltpu.VMEM((_E, 2), jnp.float32)],
    )(*([x] * _NS), W, b.reshape(_E, 1))

    rws = outs[:_NS]
    ses = outs[_NS:2 * _NS]
    loss = outs[2 * _NS]
    rw = jnp.stack([r.reshape(_K, nblk, TH) for r in rws],
                   axis=2).transpose(1, 2, 3, 0).reshape(B, S, _K)
    se = jnp.stack([s.reshape(_K, nblk, TH) for s in ses],
                   axis=2).transpose(1, 2, 3, 0).reshape(B, S, _K)
    return rw, se, loss[0, 0]


# speedup vs baseline: 1.2214x; 1.0435x over previous
import functools

import jax
import jax.numpy as jnp
from jax.experimental import pallas as pl
from jax.experimental.pallas import tpu as pltpu

_E = 64
_K = 8
_COEF = 0.01
_NS = 1


def _route_sub(x, w, b, rw_ref, se_ref):
    lt = jax.lax.dot_general(w, x, (((1,), (1,)), ((), ())),
                             preferred_element_type=jnp.float32)
    lt = lt + b
    m = jnp.max(lt, axis=0, keepdims=True)
    ex = jnp.exp(lt - m)
    scores = ex / jnp.sum(ex, axis=0, keepdims=True)
    p_part = jnp.sum(scores, axis=1, keepdims=True)

    iota = jax.lax.broadcasted_iota(jnp.int32, scores.shape, 0)
    s = scores
    vals, idxs = [], []
    for _ in range(_K):
        mk = jnp.max(s, axis=0, keepdims=True)
        ik = jnp.min(jnp.where(s == mk, iota, _E), axis=0,
                     keepdims=True)
        vals.append(mk)
        idxs.append(ik)
        s = jnp.where(iota == ik, -1.0, s)
    topv = jnp.concatenate(vals, axis=0)
    topi = jnp.concatenate(idxs, axis=0)

    mm = jnp.max(topv, axis=0, keepdims=True)
    e2 = jnp.exp(topv - mm)
    rw_ref[...] = e2 / jnp.sum(e2, axis=0, keepdims=True)
    se_ref[...] = topi

    c_part = jnp.sum((s < 0.0).astype(jnp.float32), axis=1,
                     keepdims=True)
    return p_part, c_part


def _router_body(*refs, n_tokens):
    x_refs = refs[:_NS]
    w_ref, b_ref = refs[_NS], refs[_NS + 1]
    rw_refs = refs[_NS + 2:2 * _NS + 2]
    se_refs = refs[2 * _NS + 2:3 * _NS + 2]
    loss_ref = refs[3 * _NS + 2]
    acc_ref = refs[3 * _NS + 3]

    i = pl.program_id(0)
    n = pl.num_programs(0)
    w = w_ref[...]
    b = b_ref[...]

    p_tot, c_tot = None, None
    for s in range(_NS):
        p_part, c_part = _route_sub(x_refs[s][...], w, b,
                                    rw_refs[s], se_refs[s])
        p_tot = p_part if p_tot is None else p_tot + p_part
        c_tot = c_part if c_tot is None else c_tot + c_part

    @pl.when(i == 0)
    def _init():
        acc_ref[...] = jnp.zeros_like(acc_ref)

    acc_ref[:, 0:1] += p_tot
    acc_ref[:, 1:2] += c_tot

    @pl.when(i == n - 1)
    def _fin():
        p_i = acc_ref[:, 0:1] / n_tokens
        f_i = acc_ref[:, 1:2] / (n_tokens * _K)
        loss_ref[0, 0] = _COEF * _E * jnp.sum(f_i * p_i)


def kernel(hidden_states, W, b):
    B, S, H = hidden_states.shape
    N = B * S
    x = hidden_states.reshape(N, H)
    TH = min(1024, N // _NS)
    nblk = N // (_NS * TH)

    def _in_spec(s):
        return pl.BlockSpec((TH, H), lambda i, s=s: (_NS * i + s, 0))

    outs = pl.pallas_call(
        functools.partial(_router_body, n_tokens=float(N)),
        grid=(nblk,),
        in_specs=[_in_spec(s) for s in range(_NS)] + [
            pl.BlockSpec((_E, H), lambda i: (0, 0)),
            pl.BlockSpec((_E, 1), lambda i: (0, 0)),
        ],
        out_specs=[pl.BlockSpec((_K, TH), lambda i: (0, i))
                   for _ in range(2 * _NS)] + [
            pl.BlockSpec((1, 1), lambda i: (0, 0), memory_space=pltpu.SMEM),
        ],
        out_shape=[jax.ShapeDtypeStruct((_K, N // _NS), jnp.float32)
                   for _ in range(_NS)] +
                  [jax.ShapeDtypeStruct((_K, N // _NS), jnp.int32)
                   for _ in range(_NS)] +
                  [jax.ShapeDtypeStruct((1, 1), jnp.float32)],
        scratch_shapes=[pltpu.VMEM((_E, 2), jnp.float32)],
    )(*([x] * _NS), W, b.reshape(_E, 1))

    rws = outs[:_NS]
    ses = outs[_NS:2 * _NS]
    loss = outs[2 * _NS]
    rw = jnp.stack([r.reshape(_K, nblk, TH) for r in rws],
                   axis=2).transpose(1, 2, 3, 0).reshape(B, S, _K)
    se = jnp.stack([s.reshape(_K, nblk, TH) for s in ses],
                   axis=2).transpose(1, 2, 3, 0).reshape(B, S, _K)
    return rw, se, loss[0, 0]
